# baseline (speedup 1.0000x reference)
import jax
import jax.numpy as jnp
from jax.experimental import pallas as pl

_B, _N, _NC, _L, _QC, _SC = 8, 16384, 13, 1024, 96, 64
_T = 2048
_NT = _N // _T

_CONTRACT_11 = (((1,), (1,)), ((), ()))
_CONTRACT_00 = (((0,), (0,)), ((), ()))


def _dg(a, b, dims):
    return jax.lax.dot_general(a, b, dims, preferred_element_type=jnp.float32)


def _bdot(a, b):
    return jnp.dot(a.astype(jnp.bfloat16), b.astype(jnp.bfloat16),
                   preferred_element_type=jnp.float32)


def _proto_kernel(mem_ref, wk_ref, bk_ref, wv_ref, bv_ref, bq_ref,
                  memn_ref, k_ref, v_ref, kbq_ref):
    mem = mem_ref[...]
    mean = jnp.mean(mem, axis=1)
    nrm = jnp.sqrt(jnp.sum(mean * mean, axis=-1, keepdims=True))
    memn = mean / jnp.maximum(nrm, 1e-12)
    memn_ref[...] = memn
    k = jnp.dot(memn, wk_ref[...],
                preferred_element_type=jnp.float32) + bk_ref[...]
    k_ref[...] = k
    v_ref[...] = jnp.dot(memn, wv_ref[...],
                         preferred_element_type=jnp.float32) + bv_ref[...]
    kbq_ref[...] = _dg(k, bq_ref[...], _CONTRACT_11)


def _main_kernel(x_ref, gts_ref, memn_ref, k_ref, v_ref, kbq_ref,
                 p1_ref, p2_ref, p3_ref, p3b_ref,
                 wq_ref, wo_ref, bo_ref,
                 a1_ref, a1b_ref, a2_ref, a2b_ref,
                 ones_qc_ref, ones_sc_ref,
                 out_ref, loss_ref):
    i = pl.program_id(0)

    x = x_ref[...]

    h = jnp.maximum(_bdot(x, p1_ref[...]), 0.0)
    h = jnp.maximum(_bdot(h, p2_ref[...]), 0.0)
    proj = _bdot(h, p3_ref[...]) + p3b_ref[...]
    pn2 = _dg(ones_sc_ref[...], proj * proj, _CONTRACT_11)
    rs = 1.0 / jnp.maximum(jnp.sqrt(pn2), 1e-12)
    logits = _dg(memn_ref[...], proj, _CONTRACT_11) * rs
    sumexp = jnp.sum(jnp.exp(logits), axis=0, keepdims=True)
    lse = jnp.log(sumexp)
    idx = gts_ref[0]
    mask = (jax.lax.broadcasted_iota(jnp.int32, logits.shape, 0)
            == idx).astype(jnp.float32)
    part = (jnp.sum(lse, keepdims=True)
            - jnp.sum(mask * logits, keepdims=True))

    @pl.when(i == 0)
    def _init():
        loss_ref[...] = jnp.zeros_like(loss_ref)

    loss_ref[...] += part

    xn2 = _dg(ones_qc_ref[...], x * x, _CONTRACT_11)
    rx = 1.0 / jnp.maximum(jnp.sqrt(xn2), 1e-12)
    xq = _bdot(x, wq_ref[...])
    scores = (_dg(k_ref[...], xq, _CONTRACT_11) * rx
              + kbq_ref[...]) * 0.125
    e = jnp.exp(scores)
    att = e / jnp.sum(e, axis=0, keepdims=True)
    ctx = _dg(att, v_ref[...], _CONTRACT_00)
    reve = _bdot(ctx, wo_ref[...]) + bo_ref[...]
    res = x + reve

    h2 = jnp.maximum(_bdot(res, a1_ref[...]) + a1b_ref[...], 0.0)
    out_ref[0] = res + _bdot(h2, a2_ref[...]) + a2b_ref[...]


@jax.jit
def _run(features, gts, memory, Wq, bq, Wk, bk, Wv, bv, Wo, bo,
         P1, P2, P3, p3b, A1, a1b, A2, a2b):
    memn, kmat, vmat, kbq = pl.pallas_call(
        _proto_kernel,
        out_shape=[
            jax.ShapeDtypeStruct((_NC, _SC), jnp.float32),
            jax.ShapeDtypeStruct((_NC, _SC), jnp.float32),
            jax.ShapeDtypeStruct((_NC, _SC), jnp.float32),
            jax.ShapeDtypeStruct((_NC, 1), jnp.float32),
        ],
    )(memory, Wk, bk.reshape(1, _SC), Wv, bv.reshape(1, _SC),
      bq.reshape(1, _SC))

    m_tokens = _B * _N
    ntiles = m_tokens // _T
    x2 = features.reshape(m_tokens, _QC)
    gts3 = gts.reshape(ntiles, 1, _T)
    ones_qc = jnp.ones((1, _QC), jnp.float32)
    ones_sc = jnp.ones((1, _SC), jnp.float32)

    full = lambda *s: pl.BlockSpec(s, lambda i: (0,) * len(s))
    out, loss_acc = pl.pallas_call(
        _main_kernel,
        grid=(ntiles,),
        in_specs=[
            pl.BlockSpec((_T, _QC), lambda i: (i, 0)),
            pl.BlockSpec((1, 1, _T), lambda i: (i, 0, 0)),
            full(_NC, _SC), full(_NC, _SC), full(_NC, _SC), full(_NC, 1),
            full(_QC, _SC), full(_SC, _SC), full(_SC, _SC), full(1, _SC),
            full(_QC, _SC), full(_SC, _QC), full(1, _QC),
            full(_QC, 4 * _QC), full(1, 4 * _QC),
            full(4 * _QC, _QC), full(1, _QC),
            full(1, _QC), full(1, _SC),
        ],
        out_specs=[
            pl.BlockSpec((1, _T, _QC), lambda i: (i // _NT, i % _NT, 0)),
            pl.BlockSpec((1, 1), lambda i: (0, 0)),
        ],
        out_shape=[
            jax.ShapeDtypeStruct((_B, _N, _QC), jnp.float32),
            jax.ShapeDtypeStruct((1, 1), jnp.float32),
        ],
    )(x2, gts3, memn, kmat, vmat, kbq,
      P1, P2, P3, p3b.reshape(1, _SC),
      Wq, Wo, bo.reshape(1, _QC),
      A1, a1b.reshape(1, 4 * _QC), A2, a2b.reshape(1, _QC),
      ones_qc, ones_sc)

    loss = loss_acc[0, 0] / jnp.float32(m_tokens)
    return out, loss


def kernel(features, coarse_pred, gts, memory, Wq, bq, Wk, bk, Wv, bv,
           Wo, bo, P1, P2, P3, p3b, A1, a1b, A2, a2b):
    del coarse_pred
    return _run(features, gts, memory, Wq, bq, Wk, bk, Wv, bv, Wo, bo,
                P1, P2, P3, p3b, A1, a1b, A2, a2b)


# speedup vs baseline: 1.0321x; 1.0321x over previous
import jax
import jax.numpy as jnp
from jax.experimental import pallas as pl

_B, _N, _NC, _L, _QC, _SC = 8, 16384, 13, 1024, 96, 64
_T = 2048
_NT = _N // _T

_CONTRACT_11 = (((1,), (1,)), ((), ()))
_CONTRACT_00 = (((0,), (0,)), ((), ()))


def _dg(a, b, dims):
    return jax.lax.dot_general(a, b, dims, preferred_element_type=jnp.float32)


def _bdot(a, b):
    return jnp.dot(a.astype(jnp.bfloat16), b.astype(jnp.bfloat16),
                   preferred_element_type=jnp.float32)


def _proto_kernel(mem_ref, wk_ref, bk_ref, wv_ref, bv_ref, bq_ref,
                  memn_ref, k_ref, v_ref, kbq_ref):
    mem = mem_ref[...]
    mean = jnp.mean(mem, axis=1)
    nrm = jnp.sqrt(jnp.sum(mean * mean, axis=-1, keepdims=True))
    memn = mean / jnp.maximum(nrm, 1e-12)
    memn_ref[...] = memn
    k = jnp.dot(memn, wk_ref[...],
                preferred_element_type=jnp.float32) + bk_ref[...]
    k_ref[...] = k
    v_ref[...] = jnp.dot(memn, wv_ref[...],
                         preferred_element_type=jnp.float32) + bv_ref[...]
    kbq_ref[...] = _dg(k, bq_ref[...], _CONTRACT_11)


def _main_kernel(x_ref, gts_ref, memn_ref, k_ref, v_ref, kbq_ref,
                 p1_ref, p2_ref, p3_ref, p3b_ref,
                 wq_ref, wo_ref, bo_ref,
                 a1_ref, a1b_ref, a2_ref, a2b_ref,
                 ones_qc_ref, ones_sc_ref,
                 out_ref, loss_ref):
    i = pl.program_id(0)

    x = x_ref[...]

    h = jnp.maximum(_bdot(x, p1_ref[...]), 0.0)
    h = jnp.maximum(_bdot(h, p2_ref[...]), 0.0)
    proj = _bdot(h, p3_ref[...]) + p3b_ref[...]
    pn2 = _dg(ones_sc_ref[...], proj * proj, _CONTRACT_11)
    rs = 1.0 / jnp.maximum(jnp.sqrt(pn2), 1e-12)
    logits = _dg(memn_ref[...], proj, _CONTRACT_11) * rs
    sumexp = jnp.sum(jnp.exp(logits), axis=0, keepdims=True)
    lse = jnp.log(sumexp)
    idx = gts_ref[0]
    mask = (jax.lax.broadcasted_iota(jnp.int32, logits.shape, 0)
            == idx).astype(jnp.float32)
    part = (jnp.sum(lse, keepdims=True)
            - jnp.sum(mask * logits, keepdims=True))

    @pl.when(i == 0)
    def _init():
        loss_ref[...] = jnp.zeros_like(loss_ref)

    loss_ref[...] += part

    xn2 = _dg(ones_qc_ref[...], x * x, _CONTRACT_11)
    rx = 1.0 / jnp.maximum(jnp.sqrt(xn2), 1e-12)
    xq = _bdot(x, wq_ref[...])
    scores = (_dg(k_ref[...], xq, _CONTRACT_11) * rx
              + kbq_ref[...]) * 0.125
    e = jnp.exp(scores)
    att = e / jnp.sum(e, axis=0, keepdims=True)
    ctx = _dg(att, v_ref[...], _CONTRACT_00)
    reve = _bdot(ctx, wo_ref[...]) + bo_ref[...]
    res = x + reve

    h2 = jnp.maximum(_bdot(res, a1_ref[...]) + a1b_ref[...], 0.0)
    out_ref[0, 0] = res + _bdot(h2, a2_ref[...]) + a2b_ref[...]


@jax.jit
def _run(features, gts, memory, Wq, bq, Wk, bk, Wv, bv, Wo, bo,
         P1, P2, P3, p3b, A1, a1b, A2, a2b):
    memn, kmat, vmat, kbq = pl.pallas_call(
        _proto_kernel,
        out_shape=[
            jax.ShapeDtypeStruct((_NC, _SC), jnp.float32),
            jax.ShapeDtypeStruct((_NC, _SC), jnp.float32),
            jax.ShapeDtypeStruct((_NC, _SC), jnp.float32),
            jax.ShapeDtypeStruct((_NC, 1), jnp.float32),
        ],
    )(memory, Wk, bk.reshape(1, _SC), Wv, bv.reshape(1, _SC),
      bq.reshape(1, _SC))

    m_tokens = _B * _N
    ntiles = m_tokens // _T
    x2 = features.reshape(m_tokens, _QC)
    gts3 = gts.reshape(ntiles, 1, _T)
    ones_qc = jnp.ones((1, _QC), jnp.float32)
    ones_sc = jnp.ones((1, _SC), jnp.float32)

    full = lambda *s: pl.BlockSpec(s, lambda i: (0,) * len(s))
    out, loss_acc = pl.pallas_call(
        _main_kernel,
        grid=(ntiles,),
        in_specs=[
            pl.BlockSpec((_T, _QC), lambda i: (i, 0)),
            pl.BlockSpec((1, 1, _T), lambda i: (i, 0, 0)),
            full(_NC, _SC), full(_NC, _SC), full(_NC, _SC), full(_NC, 1),
            full(_QC, _SC), full(_SC, _SC), full(_SC, _SC), full(1, _SC),
            full(_QC, _SC), full(_SC, _QC), full(1, _QC),
            full(_QC, 4 * _QC), full(1, 4 * _QC),
            full(4 * _QC, _QC), full(1, _QC),
            full(1, _QC), full(1, _SC),
        ],
        out_specs=[
            pl.BlockSpec((1, 1, _T, _QC), lambda i: (i // _NT, i % _NT, 0, 0)),
            pl.BlockSpec((1, 1), lambda i: (0, 0)),
        ],
        out_shape=[
            jax.ShapeDtypeStruct((_B, _NT, _T, _QC), jnp.float32),
            jax.ShapeDtypeStruct((1, 1), jnp.float32),
        ],
    )(x2, gts3, memn, kmat, vmat, kbq,
      P1, P2, P3, p3b.reshape(1, _SC),
      Wq, Wo, bo.reshape(1, _QC),
      A1, a1b.reshape(1, 4 * _QC), A2, a2b.reshape(1, _QC),
      ones_qc, ones_sc)

    loss = loss_acc[0, 0] / jnp.float32(m_tokens)
    return out.reshape(_B, _N, _QC), loss


def kernel(features, coarse_pred, gts, memory, Wq, bq, Wk, bk, Wv, bv,
           Wo, bo, P1, P2, P3, p3b, A1, a1b, A2, a2b):
    del coarse_pred
    return _run(features, gts, memory, Wq, bq, Wk, bk, Wv, bv, Wo, bo,
                P1, P2, P3, p3b, A1, a1b, A2, a2b)


# speedup vs baseline: 1.8514x; 1.7939x over previous
import jax
import jax.numpy as jnp
from jax.experimental import pallas as pl

_B, _N, _NC, _L, _QC, _SC = 8, 16384, 13, 1024, 96, 64
_T = 2048
_NT = _N // _T

_C11 = (((1,), (1,)), ((), ()))


def _dg(a, b, dims):
    return jax.lax.dot_general(a, b, dims, preferred_element_type=jnp.float32)


def _proto_kernel(mem_ref, wkt_ref, wvt_ref, bk_ref, bv_ref, bq_ref,
                  p3b_ref, bo_ref, a1b_ref, a2b_ref,
                  memn_ref, k_ref, vt_ref,
                  bqc_ref, p3bc_ref, boc_ref, a1bc_ref, a2bc_ref):
    mem = mem_ref[...]
    mean = jnp.sum(mem, axis=2) * (1.0 / _L)
    nrm = jnp.sqrt(jnp.sum(mean * mean, axis=-1, keepdims=True))
    memn = mean / jnp.maximum(nrm, 1e-12)
    memn_ref[...] = memn
    k_ref[...] = _dg(memn, wkt_ref[...], _C11) + bk_ref[...]
    v = _dg(memn, wvt_ref[...], _C11) + bv_ref[...]
    vt_ref[...] = v.T
    bqc_ref[...] = bq_ref[...].T
    p3bc_ref[...] = p3b_ref[...].T
    boc_ref[...] = bo_ref[...].T
    a1bc_ref[...] = a1b_ref[...].T
    a2bc_ref[...] = a2b_ref[...].T


def _main_kernel(x_ref, gts_ref, memn_ref, k_ref, vt_ref,
                 bqc_ref, p3bc_ref, boc_ref, a1bc_ref, a2bc_ref,
                 p1t_ref, p2t_ref, p3t_ref, wqt_ref, wot_ref,
                 a1t_ref, a2t_ref, ones_qc_ref, ones_sc_ref,
                 out_ref, loss_ref):
    i = pl.program_id(0)

    x = x_ref[0]

    h = jnp.maximum(jnp.dot(p1t_ref[...], x,
                            preferred_element_type=jnp.float32), 0.0)
    h = jnp.maximum(jnp.dot(p2t_ref[...], h,
                            preferred_element_type=jnp.float32), 0.0)
    proj = jnp.dot(p3t_ref[...], h,
                   preferred_element_type=jnp.float32) + p3bc_ref[...]
    pn2 = jnp.dot(ones_sc_ref[...], proj * proj,
                  preferred_element_type=jnp.float32)
    rs = 1.0 / jnp.maximum(jnp.sqrt(pn2), 1e-12)
    logits = jnp.dot(memn_ref[...], proj,
                     preferred_element_type=jnp.float32) * rs
    sumexp = jnp.sum(jnp.exp(logits), axis=0, keepdims=True)
    lse = jnp.log(sumexp)

    gts_blk = gts_ref[...]
    row = jax.lax.broadcasted_iota(jnp.int32, gts_blk.shape, 0)
    idx = jnp.sum(jnp.where(row == i // _NT, gts_blk, 0),
                  axis=0, keepdims=True)
    mask = (jax.lax.broadcasted_iota(jnp.int32, logits.shape, 0)
            == idx).astype(jnp.float32)
    part = (jnp.sum(lse, keepdims=True)
            - jnp.sum(mask * logits, keepdims=True))

    @pl.when(i == 0)
    def _init():
        loss_ref[...] = jnp.zeros_like(loss_ref)

    loss_ref[...] += part

    xn2 = jnp.dot(ones_qc_ref[...], x * x,
                  preferred_element_type=jnp.float32)
    rx = 1.0 / jnp.maximum(jnp.sqrt(xn2), 1e-12)
    q = jnp.dot(wqt_ref[...], x,
                preferred_element_type=jnp.float32) * rx + bqc_ref[...]
    scores = jnp.dot(k_ref[...], q,
                     preferred_element_type=jnp.float32) * 0.125
    e = jnp.exp(scores)
    att = e / jnp.sum(e, axis=0, keepdims=True)
    ctx = jnp.dot(vt_ref[...], att,
                  preferred_element_type=jnp.float32)
    reve = jnp.dot(wot_ref[...], ctx,
                   preferred_element_type=jnp.float32) + boc_ref[...]
    res = x + reve

    h2 = jnp.maximum(jnp.dot(a1t_ref[...], res,
                             preferred_element_type=jnp.float32)
                     + a1bc_ref[...], 0.0)
    out_ref[0] = res + jnp.dot(a2t_ref[...], h2,
                               preferred_element_type=jnp.float32) + a2bc_ref[...]


@jax.jit
def _run(features, gts, memory, Wq, bq, Wk, bk, Wv, bv, Wo, bo,
         P1, P2, P3, p3b, A1, a1b, A2, a2b):
    xT = jnp.transpose(features, (0, 2, 1))
    memT = jnp.transpose(memory, (0, 2, 1))

    memn, kmat, vt, bqc, p3bc, boc, a1bc, a2bc = pl.pallas_call(
        _proto_kernel,
        out_shape=[
            jax.ShapeDtypeStruct((_NC, _SC), jnp.float32),
            jax.ShapeDtypeStruct((_NC, _SC), jnp.float32),
            jax.ShapeDtypeStruct((_SC, _NC), jnp.float32),
            jax.ShapeDtypeStruct((_SC, 1), jnp.float32),
            jax.ShapeDtypeStruct((_SC, 1), jnp.float32),
            jax.ShapeDtypeStruct((_QC, 1), jnp.float32),
            jax.ShapeDtypeStruct((4 * _QC, 1), jnp.float32),
            jax.ShapeDtypeStruct((_QC, 1), jnp.float32),
        ],
    )(memT, Wk.T, Wv.T, bk.reshape(1, _SC), bv.reshape(1, _SC),
      bq.reshape(1, _SC), p3b.reshape(1, _SC), bo.reshape(1, _QC),
      a1b.reshape(1, 4 * _QC), a2b.reshape(1, _QC))

    ones_qc = jnp.ones((1, _QC), jnp.float32)
    ones_sc = jnp.ones((1, _SC), jnp.float32)

    full = lambda *s: pl.BlockSpec(s, lambda i: (0,) * len(s))
    outT, loss_acc = pl.pallas_call(
        _main_kernel,
        grid=(_B * _NT,),
        in_specs=[
            pl.BlockSpec((1, _QC, _T), lambda i: (i // _NT, 0, i % _NT)),
            pl.BlockSpec((_B, _T), lambda i: (0, i % _NT)),
            full(_NC, _SC), full(_NC, _SC), full(_SC, _NC),
            full(_SC, 1), full(_SC, 1), full(_QC, 1),
            full(4 * _QC, 1), full(_QC, 1),
            full(_SC, _QC), full(_SC, _SC), full(_SC, _SC),
            full(_SC, _QC), full(_QC, _SC),
            full(4 * _QC, _QC), full(_QC, 4 * _QC),
            full(1, _QC), full(1, _SC),
        ],
        out_specs=[
            pl.BlockSpec((1, _QC, _T), lambda i: (i // _NT, 0, i % _NT)),
            pl.BlockSpec((1, 1), lambda i: (0, 0)),
        ],
        out_shape=[
            jax.ShapeDtypeStruct((_B, _QC, _N), jnp.float32),
            jax.ShapeDtypeStruct((1, 1), jnp.float32),
        ],
    )(xT, gts, memn, kmat, vt, bqc, p3bc, boc, a1bc, a2bc,
      P1.T, P2.T, P3.T, Wq.T, Wo.T, A1.T, A2.T,
      ones_qc, ones_sc)

    res = jnp.transpose(outT, (0, 2, 1))
    loss = loss_acc[0, 0] / jnp.float32(_B * _N)
    return res, loss


def kernel(features, coarse_pred, gts, memory, Wq, bq, Wk, bk, Wv, bv,
           Wo, bo, P1, P2, P3, p3b, A1, a1b, A2, a2b):
    del coarse_pred
    return _run(features, gts, memory, Wq, bq, Wk, bk, Wv, bv, Wo, bo,
                P1, P2, P3, p3b, A1, a1b, A2, a2b)


# speedup vs baseline: 2.5388x; 1.3713x over previous
import jax
import jax.numpy as jnp
from jax.experimental import pallas as pl

_B, _N, _NC, _L, _QC, _SC = 8, 16384, 13, 1024, 96, 64
_T = 4096
_NT = _N // _T

_C11 = (((1,), (1,)), ((), ()))


def _dg(a, b, dims):
    return jax.lax.dot_general(a, b, dims, preferred_element_type=jnp.float32)


def _proto_kernel(mem_ref, wkt_ref, wvt_ref, bk_ref, bv_ref, bq_ref,
                  p3b_ref, bo_ref, a1b_ref, a2b_ref,
                  memn_ref, k_ref, vt_ref,
                  bqc_ref, p3bc_ref, boc_ref, a1bc_ref, a2bc_ref):
    mem = mem_ref[...]
    mean = jnp.sum(mem, axis=2) * (1.0 / _L)
    nrm = jnp.sqrt(jnp.sum(mean * mean, axis=-1, keepdims=True))
    memn = mean / jnp.maximum(nrm, 1e-12)
    memn_ref[...] = memn
    k_ref[...] = _dg(memn, wkt_ref[...], _C11) + bk_ref[...]
    v = _dg(memn, wvt_ref[...], _C11) + bv_ref[...]
    vt_ref[...] = v.T
    bqc_ref[...] = bq_ref[...].T
    p3bc_ref[...] = p3b_ref[...].T
    boc_ref[...] = bo_ref[...].T
    a1bc_ref[...] = a1b_ref[...].T
    a2bc_ref[...] = a2b_ref[...].T


def _main_kernel(x_ref, gts_ref, memn_ref, k_ref, vt_ref,
                 bqc_ref, p3bc_ref, boc_ref, a1bc_ref, a2bc_ref,
                 p1t_ref, p2t_ref, p3t_ref, wqt_ref, wot_ref,
                 a1t_ref, a2t_ref, ones_qc_ref, ones_sc_ref,
                 out_ref, loss_ref):
    i = pl.program_id(0)

    x = x_ref[0]

    h = jnp.maximum(jnp.dot(p1t_ref[...], x,
                            preferred_element_type=jnp.float32), 0.0)
    h = jnp.maximum(jnp.dot(p2t_ref[...], h,
                            preferred_element_type=jnp.float32), 0.0)
    proj = jnp.dot(p3t_ref[...], h,
                   preferred_element_type=jnp.float32) + p3bc_ref[...]
    pn2 = jnp.dot(ones_sc_ref[...], proj * proj,
                  preferred_element_type=jnp.float32)
    rs = 1.0 / jnp.maximum(jnp.sqrt(pn2), 1e-12)
    logits = jnp.dot(memn_ref[...], proj,
                     preferred_element_type=jnp.float32) * rs
    sumexp = jnp.sum(jnp.exp(logits), axis=0, keepdims=True)
    lse = jnp.log(sumexp)

    gts_blk = gts_ref[...]
    row = jax.lax.broadcasted_iota(jnp.int32, gts_blk.shape, 0)
    idx = jnp.sum(jnp.where(row == i // _NT, gts_blk, 0),
                  axis=0, keepdims=True)
    mask = (jax.lax.broadcasted_iota(jnp.int32, logits.shape, 0)
            == idx).astype(jnp.float32)
    part = (jnp.sum(lse, keepdims=True)
            - jnp.sum(mask * logits, keepdims=True))

    @pl.when(i == 0)
    def _init():
        loss_ref[...] = jnp.zeros_like(loss_ref)

    loss_ref[...] += part

    xn2 = jnp.dot(ones_qc_ref[...], x * x,
                  preferred_element_type=jnp.float32)
    rx = 1.0 / jnp.maximum(jnp.sqrt(xn2), 1e-12)
    q = jnp.dot(wqt_ref[...], x,
                preferred_element_type=jnp.float32) * rx + bqc_ref[...]
    scores = jnp.dot(k_ref[...], q,
                     preferred_element_type=jnp.float32) * 0.125
    e = jnp.exp(scores)
    att = e / jnp.sum(e, axis=0, keepdims=True)
    ctx = jnp.dot(vt_ref[...], att,
                  preferred_element_type=jnp.float32)
    reve = jnp.dot(wot_ref[...], ctx,
                   preferred_element_type=jnp.float32) + boc_ref[...]
    res = x + reve

    h2 = jnp.maximum(jnp.dot(a1t_ref[...], res,
                             preferred_element_type=jnp.float32)
                     + a1bc_ref[...], 0.0)
    out_ref[0] = res + jnp.dot(a2t_ref[...], h2,
                               preferred_element_type=jnp.float32) + a2bc_ref[...]


@jax.jit
def _run(features, gts, memory, Wq, bq, Wk, bk, Wv, bv, Wo, bo,
         P1, P2, P3, p3b, A1, a1b, A2, a2b):
    xT = jnp.transpose(features, (0, 2, 1))
    memT = jnp.transpose(memory, (0, 2, 1))

    memn, kmat, vt, bqc, p3bc, boc, a1bc, a2bc = pl.pallas_call(
        _proto_kernel,
        out_shape=[
            jax.ShapeDtypeStruct((_NC, _SC), jnp.float32),
            jax.ShapeDtypeStruct((_NC, _SC), jnp.float32),
            jax.ShapeDtypeStruct((_SC, _NC), jnp.float32),
            jax.ShapeDtypeStruct((_SC, 1), jnp.float32),
            jax.ShapeDtypeStruct((_SC, 1), jnp.float32),
            jax.ShapeDtypeStruct((_QC, 1), jnp.float32),
            jax.ShapeDtypeStruct((4 * _QC, 1), jnp.float32),
            jax.ShapeDtypeStruct((_QC, 1), jnp.float32),
        ],
    )(memT, Wk.T, Wv.T, bk.reshape(1, _SC), bv.reshape(1, _SC),
      bq.reshape(1, _SC), p3b.reshape(1, _SC), bo.reshape(1, _QC),
      a1b.reshape(1, 4 * _QC), a2b.reshape(1, _QC))

    ones_qc = jnp.ones((1, _QC), jnp.float32)
    ones_sc = jnp.ones((1, _SC), jnp.float32)

    full = lambda *s: pl.BlockSpec(s, lambda i: (0,) * len(s))
    outT, loss_acc = pl.pallas_call(
        _main_kernel,
        grid=(_B * _NT,),
        in_specs=[
            pl.BlockSpec((1, _QC, _T), lambda i: (i // _NT, 0, i % _NT)),
            pl.BlockSpec((_B, _T), lambda i: (0, i % _NT)),
            full(_NC, _SC), full(_NC, _SC), full(_SC, _NC),
            full(_SC, 1), full(_SC, 1), full(_QC, 1),
            full(4 * _QC, 1), full(_QC, 1),
            full(_SC, _QC), full(_SC, _SC), full(_SC, _SC),
            full(_SC, _QC), full(_QC, _SC),
            full(4 * _QC, _QC), full(_QC, 4 * _QC),
            full(1, _QC), full(1, _SC),
        ],
        out_specs=[
            pl.BlockSpec((1, _QC, _T), lambda i: (i // _NT, 0, i % _NT)),
            pl.BlockSpec((1, 1), lambda i: (0, 0)),
        ],
        out_shape=[
            jax.ShapeDtypeStruct((_B, _QC, _N), jnp.float32),
            jax.ShapeDtypeStruct((1, 1), jnp.float32),
        ],
    )(xT, gts, memn, kmat, vt, bqc, p3bc, boc, a1bc, a2bc,
      P1.T, P2.T, P3.T, Wq.T, Wo.T, A1.T, A2.T,
      ones_qc, ones_sc)

    res = jnp.transpose(outT, (0, 2, 1))
    loss = loss_acc[0, 0] / jnp.float32(_B * _N)
    return res, loss


def kernel(features, coarse_pred, gts, memory, Wq, bq, Wk, bk, Wv, bv,
           Wo, bo, P1, P2, P3, p3b, A1, a1b, A2, a2b):
    del coarse_pred
    return _run(features, gts, memory, Wq, bq, Wk, bk, Wv, bv, Wo, bo,
                P1, P2, P3, p3b, A1, a1b, A2, a2b)


# speedup vs baseline: 2.7798x; 1.0949x over previous
import jax
import jax.numpy as jnp
from jax.experimental import pallas as pl

_B, _N, _NC, _L, _QC, _SC = 8, 16384, 13, 1024, 96, 64
_T = 8192
_NT = _N // _T

_C11 = (((1,), (1,)), ((), ()))


def _dg(a, b, dims):
    return jax.lax.dot_general(a, b, dims, preferred_element_type=jnp.float32)


def _proto_kernel(mem_ref, wkt_ref, wvt_ref, bk_ref, bv_ref, bq_ref,
                  p3b_ref, bo_ref, a1b_ref, a2b_ref,
                  memn_ref, k_ref, vt_ref,
                  bqc_ref, p3bc_ref, boc_ref, a1bc_ref, a2bc_ref):
    mem = mem_ref[...]
    mean = jnp.sum(mem, axis=2) * (1.0 / _L)
    nrm = jnp.sqrt(jnp.sum(mean * mean, axis=-1, keepdims=True))
    memn = mean / jnp.maximum(nrm, 1e-12)
    memn_ref[...] = memn
    k_ref[...] = _dg(memn, wkt_ref[...], _C11) + bk_ref[...]
    v = _dg(memn, wvt_ref[...], _C11) + bv_ref[...]
    vt_ref[...] = v.T
    bqc_ref[...] = bq_ref[...].T
    p3bc_ref[...] = p3b_ref[...].T
    boc_ref[...] = bo_ref[...].T
    a1bc_ref[...] = a1b_ref[...].T
    a2bc_ref[...] = a2b_ref[...].T


def _main_kernel(x_ref, gts_ref, memn_ref, k_ref, vt_ref,
                 bqc_ref, p3bc_ref, boc_ref, a1bc_ref, a2bc_ref,
                 p1t_ref, p2t_ref, p3t_ref, wqt_ref, wot_ref,
                 a1t_ref, a2t_ref, ones_qc_ref, ones_sc_ref,
                 out_ref, loss_ref):
    i = pl.program_id(0)

    x = x_ref[0]

    h = jnp.maximum(jnp.dot(p1t_ref[...], x,
                            preferred_element_type=jnp.float32), 0.0)
    h = jnp.maximum(jnp.dot(p2t_ref[...], h,
                            preferred_element_type=jnp.float32), 0.0)
    proj = jnp.dot(p3t_ref[...], h,
                   preferred_element_type=jnp.float32) + p3bc_ref[...]
    pn2 = jnp.dot(ones_sc_ref[...], proj * proj,
                  preferred_element_type=jnp.float32)
    rs = 1.0 / jnp.maximum(jnp.sqrt(pn2), 1e-12)
    logits = jnp.dot(memn_ref[...], proj,
                     preferred_element_type=jnp.float32) * rs
    sumexp = jnp.sum(jnp.exp(logits), axis=0, keepdims=True)
    lse = jnp.log(sumexp)

    gts_blk = gts_ref[...]
    row = jax.lax.broadcasted_iota(jnp.int32, gts_blk.shape, 0)
    idx = jnp.sum(jnp.where(row == i // _NT, gts_blk, 0),
                  axis=0, keepdims=True)
    mask = (jax.lax.broadcasted_iota(jnp.int32, logits.shape, 0)
            == idx).astype(jnp.float32)
    part = (jnp.sum(lse, keepdims=True)
            - jnp.sum(mask * logits, keepdims=True))

    @pl.when(i == 0)
    def _init():
        loss_ref[...] = jnp.zeros_like(loss_ref)

    loss_ref[...] += part

    xn2 = jnp.dot(ones_qc_ref[...], x * x,
                  preferred_element_type=jnp.float32)
    rx = 1.0 / jnp.maximum(jnp.sqrt(xn2), 1e-12)
    q = jnp.dot(wqt_ref[...], x,
                preferred_element_type=jnp.float32) * rx + bqc_ref[...]
    scores = jnp.dot(k_ref[...], q,
                     preferred_element_type=jnp.float32) * 0.125
    e = jnp.exp(scores)
    att = e / jnp.sum(e, axis=0, keepdims=True)
    ctx = jnp.dot(vt_ref[...], att,
                  preferred_element_type=jnp.float32)
    reve = jnp.dot(wot_ref[...], ctx,
                   preferred_element_type=jnp.float32) + boc_ref[...]
    res = x + reve

    h2 = jnp.maximum(jnp.dot(a1t_ref[...], res,
                             preferred_element_type=jnp.float32)
                     + a1bc_ref[...], 0.0)
    out_ref[0] = res + jnp.dot(a2t_ref[...], h2,
                               preferred_element_type=jnp.float32) + a2bc_ref[...]


@jax.jit
def _run(features, gts, memory, Wq, bq, Wk, bk, Wv, bv, Wo, bo,
         P1, P2, P3, p3b, A1, a1b, A2, a2b):
    xT = jnp.transpose(features, (0, 2, 1))
    memT = jnp.transpose(memory, (0, 2, 1))

    memn, kmat, vt, bqc, p3bc, boc, a1bc, a2bc = pl.pallas_call(
        _proto_kernel,
        out_shape=[
            jax.ShapeDtypeStruct((_NC, _SC), jnp.float32),
            jax.ShapeDtypeStruct((_NC, _SC), jnp.float32),
            jax.ShapeDtypeStruct((_SC, _NC), jnp.float32),
            jax.ShapeDtypeStruct((_SC, 1), jnp.float32),
            jax.ShapeDtypeStruct((_SC, 1), jnp.float32),
            jax.ShapeDtypeStruct((_QC, 1), jnp.float32),
            jax.ShapeDtypeStruct((4 * _QC, 1), jnp.float32),
            jax.ShapeDtypeStruct((_QC, 1), jnp.float32),
        ],
    )(memT, Wk.T, Wv.T, bk.reshape(1, _SC), bv.reshape(1, _SC),
      bq.reshape(1, _SC), p3b.reshape(1, _SC), bo.reshape(1, _QC),
      a1b.reshape(1, 4 * _QC), a2b.reshape(1, _QC))

    ones_qc = jnp.ones((1, _QC), jnp.float32)
    ones_sc = jnp.ones((1, _SC), jnp.float32)

    full = lambda *s: pl.BlockSpec(s, lambda i: (0,) * len(s))
    outT, loss_acc = pl.pallas_call(
        _main_kernel,
        grid=(_B * _NT,),
        in_specs=[
            pl.BlockSpec((1, _QC, _T), lambda i: (i // _NT, 0, i % _NT)),
            pl.BlockSpec((_B, _T), lambda i: (0, i % _NT)),
            full(_NC, _SC), full(_NC, _SC), full(_SC, _NC),
            full(_SC, 1), full(_SC, 1), full(_QC, 1),
            full(4 * _QC, 1), full(_QC, 1),
            full(_SC, _QC), full(_SC, _SC), full(_SC, _SC),
            full(_SC, _QC), full(_QC, _SC),
            full(4 * _QC, _QC), full(_QC, 4 * _QC),
            full(1, _QC), full(1, _SC),
        ],
        out_specs=[
            pl.BlockSpec((1, _QC, _T), lambda i: (i // _NT, 0, i % _NT)),
            pl.BlockSpec((1, 1), lambda i: (0, 0)),
        ],
        out_shape=[
            jax.ShapeDtypeStruct((_B, _QC, _N), jnp.float32),
            jax.ShapeDtypeStruct((1, 1), jnp.float32),
        ],
    )(xT, gts, memn, kmat, vt, bqc, p3bc, boc, a1bc, a2bc,
      P1.T, P2.T, P3.T, Wq.T, Wo.T, A1.T, A2.T,
      ones_qc, ones_sc)

    res = jnp.transpose(outT, (0, 2, 1))
    loss = loss_acc[0, 0] / jnp.float32(_B * _N)
    return res, loss


def kernel(features, coarse_pred, gts, memory, Wq, bq, Wk, bk, Wv, bv,
           Wo, bo, P1, P2, P3, p3b, A1, a1b, A2, a2b):
    del coarse_pred
    return _run(features, gts, memory, Wq, bq, Wk, bk, Wv, bv, Wo, bo,
                P1, P2, P3, p3b, A1, a1b, A2, a2b)


# speedup vs baseline: 2.8484x; 1.0247x over previous
import jax
import jax.numpy as jnp
from jax.experimental import pallas as pl

_B, _N, _NC, _L, _QC, _SC = 8, 16384, 13, 1024, 96, 64
_T = 16384
_NT = _N // _T

_C11 = (((1,), (1,)), ((), ()))


def _dg(a, b, dims):
    return jax.lax.dot_general(a, b, dims, preferred_element_type=jnp.float32)


def _proto_kernel(mem_ref, wkt_ref, wvt_ref, bk_ref, bv_ref, bq_ref,
                  p3b_ref, bo_ref, a1b_ref, a2b_ref,
                  memn_ref, k_ref, vt_ref,
                  bqc_ref, p3bc_ref, boc_ref, a1bc_ref, a2bc_ref):
    mem = mem_ref[...]
    mean = jnp.sum(mem, axis=2) * (1.0 / _L)
    nrm = jnp.sqrt(jnp.sum(mean * mean, axis=-1, keepdims=True))
    memn = mean / jnp.maximum(nrm, 1e-12)
    memn_ref[...] = memn
    k_ref[...] = _dg(memn, wkt_ref[...], _C11) + bk_ref[...]
    v = _dg(memn, wvt_ref[...], _C11) + bv_ref[...]
    vt_ref[...] = v.T
    bqc_ref[...] = bq_ref[...].T
    p3bc_ref[...] = p3b_ref[...].T
    boc_ref[...] = bo_ref[...].T
    a1bc_ref[...] = a1b_ref[...].T
    a2bc_ref[...] = a2b_ref[...].T


def _main_kernel(x_ref, gts_ref, memn_ref, k_ref, vt_ref,
                 bqc_ref, p3bc_ref, boc_ref, a1bc_ref, a2bc_ref,
                 p1t_ref, p2t_ref, p3t_ref, wqt_ref, wot_ref,
                 a1t_ref, a2t_ref, ones_qc_ref, ones_sc_ref,
                 out_ref, loss_ref):
    i = pl.program_id(0)

    x = x_ref[0]

    h = jnp.maximum(jnp.dot(p1t_ref[...], x,
                            preferred_element_type=jnp.float32), 0.0)
    h = jnp.maximum(jnp.dot(p2t_ref[...], h,
                            preferred_element_type=jnp.float32), 0.0)
    proj = jnp.dot(p3t_ref[...], h,
                   preferred_element_type=jnp.float32) + p3bc_ref[...]
    pn2 = jnp.dot(ones_sc_ref[...], proj * proj,
                  preferred_element_type=jnp.float32)
    rs = 1.0 / jnp.maximum(jnp.sqrt(pn2), 1e-12)
    logits = jnp.dot(memn_ref[...], proj,
                     preferred_element_type=jnp.float32) * rs
    sumexp = jnp.sum(jnp.exp(logits), axis=0, keepdims=True)
    lse = jnp.log(sumexp)

    gts_blk = gts_ref[...]
    row = jax.lax.broadcasted_iota(jnp.int32, gts_blk.shape, 0)
    idx = jnp.sum(jnp.where(row == i // _NT, gts_blk, 0),
                  axis=0, keepdims=True)
    mask = (jax.lax.broadcasted_iota(jnp.int32, logits.shape, 0)
            == idx).astype(jnp.float32)
    part = (jnp.sum(lse, keepdims=True)
            - jnp.sum(mask * logits, keepdims=True))

    @pl.when(i == 0)
    def _init():
        loss_ref[...] = jnp.zeros_like(loss_ref)

    loss_ref[...] += part

    xn2 = jnp.dot(ones_qc_ref[...], x * x,
                  preferred_element_type=jnp.float32)
    rx = 1.0 / jnp.maximum(jnp.sqrt(xn2), 1e-12)
    q = jnp.dot(wqt_ref[...], x,
                preferred_element_type=jnp.float32) * rx + bqc_ref[...]
    scores = jnp.dot(k_ref[...], q,
                     preferred_element_type=jnp.float32) * 0.125
    e = jnp.exp(scores)
    att = e / jnp.sum(e, axis=0, keepdims=True)
    ctx = jnp.dot(vt_ref[...], att,
                  preferred_element_type=jnp.float32)
    reve = jnp.dot(wot_ref[...], ctx,
                   preferred_element_type=jnp.float32) + boc_ref[...]
    res = x + reve

    h2 = jnp.maximum(jnp.dot(a1t_ref[...], res,
                             preferred_element_type=jnp.float32)
                     + a1bc_ref[...], 0.0)
    out_ref[0] = res + jnp.dot(a2t_ref[...], h2,
                               preferred_element_type=jnp.float32) + a2bc_ref[...]


@jax.jit
def _run(features, gts, memory, Wq, bq, Wk, bk, Wv, bv, Wo, bo,
         P1, P2, P3, p3b, A1, a1b, A2, a2b):
    xT = jnp.transpose(features, (0, 2, 1))
    memT = jnp.transpose(memory, (0, 2, 1))

    memn, kmat, vt, bqc, p3bc, boc, a1bc, a2bc = pl.pallas_call(
        _proto_kernel,
        out_shape=[
            jax.ShapeDtypeStruct((_NC, _SC), jnp.float32),
            jax.ShapeDtypeStruct((_NC, _SC), jnp.float32),
            jax.ShapeDtypeStruct((_SC, _NC), jnp.float32),
            jax.ShapeDtypeStruct((_SC, 1), jnp.float32),
            jax.ShapeDtypeStruct((_SC, 1), jnp.float32),
            jax.ShapeDtypeStruct((_QC, 1), jnp.float32),
            jax.ShapeDtypeStruct((4 * _QC, 1), jnp.float32),
            jax.ShapeDtypeStruct((_QC, 1), jnp.float32),
        ],
    )(memT, Wk.T, Wv.T, bk.reshape(1, _SC), bv.reshape(1, _SC),
      bq.reshape(1, _SC), p3b.reshape(1, _SC), bo.reshape(1, _QC),
      a1b.reshape(1, 4 * _QC), a2b.reshape(1, _QC))

    ones_qc = jnp.ones((1, _QC), jnp.float32)
    ones_sc = jnp.ones((1, _SC), jnp.float32)

    full = lambda *s: pl.BlockSpec(s, lambda i: (0,) * len(s))
    outT, loss_acc = pl.pallas_call(
        _main_kernel,
        grid=(_B * _NT,),
        in_specs=[
            pl.BlockSpec((1, _QC, _T), lambda i: (i // _NT, 0, i % _NT)),
            pl.BlockSpec((_B, _T), lambda i: (0, i % _NT)),
            full(_NC, _SC), full(_NC, _SC), full(_SC, _NC),
            full(_SC, 1), full(_SC, 1), full(_QC, 1),
            full(4 * _QC, 1), full(_QC, 1),
            full(_SC, _QC), full(_SC, _SC), full(_SC, _SC),
            full(_SC, _QC), full(_QC, _SC),
            full(4 * _QC, _QC), full(_QC, 4 * _QC),
            full(1, _QC), full(1, _SC),
        ],
        out_specs=[
            pl.BlockSpec((1, _QC, _T), lambda i: (i // _NT, 0, i % _NT)),
            pl.BlockSpec((1, 1), lambda i: (0, 0)),
        ],
        out_shape=[
            jax.ShapeDtypeStruct((_B, _QC, _N), jnp.float32),
            jax.ShapeDtypeStruct((1, 1), jnp.float32),
        ],
    )(xT, gts, memn, kmat, vt, bqc, p3bc, boc, a1bc, a2bc,
      P1.T, P2.T, P3.T, Wq.T, Wo.T, A1.T, A2.T,
      ones_qc, ones_sc)

    res = jnp.transpose(outT, (0, 2, 1))
    loss = loss_acc[0, 0] / jnp.float32(_B * _N)
    return res, loss


def kernel(features, coarse_pred, gts, memory, Wq, bq, Wk, bk, Wv, bv,
           Wo, bo, P1, P2, P3, p3b, A1, a1b, A2, a2b):
    del coarse_pred
    return _run(features, gts, memory, Wq, bq, Wk, bk, Wv, bv, Wo, bo,
                P1, P2, P3, p3b, A1, a1b, A2, a2b)
